# baseline (speedup 1.0000x reference)
import functools

import jax
import jax.numpy as jnp
from jax import lax
from jax.experimental import pallas as pl
from jax.experimental.pallas import tpu as pltpu
from jax.experimental.pallas import tpu_sc as plsc

B = 16384
D = 64
NC = 2
NS = 16
NW = NC * NS
BPW = B // NW
CH = 128
NCH = BPW // CH
L = 16


def _make_kernel():
    mesh = plsc.VectorSubcoreMesh(core_axis_name="c", subcore_axis_name="s")

    @functools.partial(
        pl.kernel,
        mesh=mesh,
        compiler_params=pltpu.CompilerParams(needs_layout_passes=False,
                                             use_tc_tiling_on_sc=False),
        out_type=jax.ShapeDtypeStruct((B,), jnp.float32),
        scratch_types=[
            pltpu.VMEM((NCH, CH), jnp.int32),
            pltpu.VMEM((NCH, CH), jnp.int32),
            pltpu.VMEM((BPW, D), jnp.float32),
            pltpu.VMEM((BPW, D), jnp.float32),
            pltpu.VMEM((BPW,), jnp.float32),
            pltpu.SemaphoreType.DMA,
        ],
    )
    def dot_kernel(ent_hbm, tid_hbm, eemb_hbm, temb_hbm, out_hbm,
                   eidx, tidx, erows, trows, outv, sem):
        wid = lax.axis_index("s") * NC + lax.axis_index("c")
        base = wid * BPW

        for c in range(NCH):
            pltpu.sync_copy(ent_hbm.at[pl.ds(base + c * CH, CH)], eidx.at[c])
            pltpu.sync_copy(tid_hbm.at[pl.ds(base + c * CH, CH)], tidx.at[c])

        cps = []
        for c in range(NCH):
            cps.append(pltpu.async_copy(
                eemb_hbm.at[eidx.at[c]], erows.at[pl.ds(c * CH, CH), :], sem))
            cps.append(pltpu.async_copy(
                temb_hbm.at[tidx.at[c]], trows.at[pl.ds(c * CH, CH), :], sem))
        for cp in cps:
            cp.wait()

        lane = lax.iota(jnp.int32, L)

        def group_body(g, _):
            rbase = g * L
            acc = jnp.zeros((L,), jnp.float32)
            for i in range(L):
                r = rbase + i
                p = erows[r, pl.ds(0, L)] * trows[r, pl.ds(0, L)]
                for k in range(1, D // L):
                    p = p + erows[r, pl.ds(k * L, L)] * trows[r, pl.ds(k * L, L)]
                acc = jnp.where(lane == i, jnp.sum(p), acc)
            outv[pl.ds(rbase, L)] = acc
            return 0

        lax.fori_loop(0, BPW // L, group_body, 0)

        pltpu.sync_copy(outv, out_hbm.at[pl.ds(base, BPW)])

    return dot_kernel


_dot_kernel = _make_kernel()


def kernel(ent, ent_type, batch_type, ent_emb, type_emb):
    del batch_type
    score = _dot_kernel(ent.astype(jnp.int32), ent_type.astype(jnp.int32),
                        ent_emb, type_emb)
    return score[:, None]
